# baseline (speedup 1.0000x reference)
import functools

import jax
import jax.numpy as jnp
from jax import lax
from jax.experimental import pallas as pl
from jax.experimental.pallas import tpu as pltpu
from jax.experimental.pallas import tpu_sc as plsc

_NC = 2
_NS = 16
_NW = _NC * _NS

_CHUNK = 40
_NBUF = 10
_B_SC = 122880


def _make_gather(V, D, B, b_sc):
    b_per_w = b_sc // _NW
    n_chunks = b_per_w // _CHUNK
    n_main = (n_chunks // _NBUF - 1) * _NBUF
    tail = n_chunks - n_main - _NBUF
    mesh = plsc.VectorSubcoreMesh(core_axis_name="c", subcore_axis_name="s")

    @functools.partial(
        pl.kernel,
        mesh=mesh,
        out_type=jax.ShapeDtypeStruct((B, D), jnp.float32),
        scratch_types=[
            pltpu.VMEM((n_chunks, _CHUNK), jnp.int32),
        ] + [pltpu.VMEM((_CHUNK, D), jnp.float32)] * _NBUF
          + [pltpu.SemaphoreType.DMA] * (2 * _NBUF),
    )
    def gather_kernel(table_hbm, idx_hbm, out_hbm, idx_v, *rest):
        bufs = rest[:_NBUF]
        gsems = rest[_NBUF:2 * _NBUF]
        ssems = rest[2 * _NBUF:]
        wid = lax.axis_index("c") * _NS + lax.axis_index("s")
        base = wid * b_per_w
        pltpu.sync_copy(idx_hbm.at[wid], idx_v)

        def gather_start(j, b):
            pltpu.async_copy(table_hbm.at[idx_v.at[j]], bufs[b], gsems[b])

        def gather_wait(b):
            pltpu.make_async_copy(
                table_hbm.at[idx_v.at[0]], bufs[b], gsems[b]).wait()

        def scatter_start(j, b):
            dst = out_hbm.at[pl.ds(base + j * _CHUNK, _CHUNK)]
            pltpu.async_copy(bufs[b], dst, ssems[b])

        def scatter_wait(b):
            dst = out_hbm.at[pl.ds(base, _CHUNK)]
            pltpu.make_async_copy(bufs[b], dst, ssems[b]).wait()

        for b in range(_NBUF):
            gather_start(b, b)

        def body(i, carry):
            g = _NBUF * i
            for b in range(_NBUF):
                gather_wait(b)
                scatter_start(g + b, b)
            for b in range(_NBUF):
                scatter_wait(b)
                gather_start(g + _NBUF + b, b)
            return carry

        lax.fori_loop(0, n_main // _NBUF, body, 0)

        g = n_main
        for b in range(_NBUF):
            gather_wait(b)
            scatter_start(g + b, b)
        for b in range(tail):
            scatter_wait(b)
            gather_start(g + _NBUF + b, b)
        for b in range(tail):
            gather_wait(b)
            scatter_start(g + _NBUF + b, b)
        for b in range(_NBUF):
            scatter_wait(b)

    return gather_kernel


def kernel(tensor, idx):
    V, D = tensor.shape
    (B,) = idx.shape
    idx3 = idx[:_B_SC].reshape(_NW, _B_SC // _NW // _CHUNK, _CHUNK)
    sc_out = _make_gather(V, D, B, _B_SC)(tensor, idx3)
    tc_part = jnp.take(tensor, idx[_B_SC:], axis=0)
    return lax.dynamic_update_slice(sc_out, tc_part, (_B_SC, 0))


# speedup vs baseline: 1.2363x; 1.2363x over previous
import functools

import jax
import jax.numpy as jnp
from jax import lax
from jax.experimental import pallas as pl
from jax.experimental.pallas import tpu as pltpu
from jax.experimental.pallas import tpu_sc as plsc

_NC = 2
_NS = 16
_NW = _NC * _NS

_CHUNK = 40
_NBUF = 5


def _make_gather(V, D, B):
    b_per_w = B // _NW
    n_chunks = b_per_w // _CHUNK
    n_main = (n_chunks // _NBUF - 1) * _NBUF
    tail = n_chunks - n_main - _NBUF
    mesh = plsc.VectorSubcoreMesh(core_axis_name="c", subcore_axis_name="s")

    @functools.partial(
        pl.kernel,
        mesh=mesh,
        out_type=jax.ShapeDtypeStruct((B, D), jnp.float32),
        scratch_types=[
            pltpu.VMEM((n_chunks, _CHUNK), jnp.int32),
        ] + [pltpu.VMEM((_CHUNK, D), jnp.float32)] * _NBUF
          + [pltpu.SemaphoreType.DMA] * (2 * _NBUF),
    )
    def gather_kernel(table_hbm, idx_hbm, out_hbm, idx_v, *rest):
        bufs = rest[:_NBUF]
        gsems = rest[_NBUF:2 * _NBUF]
        ssems = rest[2 * _NBUF:]
        wid = lax.axis_index("s") * _NC + lax.axis_index("c")
        base = wid * b_per_w
        pltpu.sync_copy(idx_hbm.at[wid], idx_v)

        def gather_start(j, b):
            pltpu.async_copy(table_hbm.at[idx_v.at[j]], bufs[b], gsems[b])

        def gather_wait(b):
            pltpu.make_async_copy(
                table_hbm.at[idx_v.at[0]], bufs[b], gsems[b]).wait()

        def scatter_start(j, b):
            dst = out_hbm.at[pl.ds(base + j * _CHUNK, _CHUNK)]
            pltpu.async_copy(bufs[b], dst, ssems[b])

        def scatter_wait(b):
            dst = out_hbm.at[pl.ds(base, _CHUNK)]
            pltpu.make_async_copy(bufs[b], dst, ssems[b]).wait()

        for b in range(_NBUF):
            gather_start(b, b)

        def body(i, carry):
            g = _NBUF * i
            for b in range(_NBUF):
                gather_wait(b)
                scatter_start(g + b, b)
            for b in range(_NBUF):
                scatter_wait(b)
                gather_start(g + _NBUF + b, b)
            return carry

        lax.fori_loop(0, n_main // _NBUF, body, 0)

        g = n_main
        for b in range(_NBUF):
            gather_wait(b)
            scatter_start(g + b, b)
        for b in range(tail):
            scatter_wait(b)
            gather_start(g + _NBUF + b, b)
        for b in range(tail):
            gather_wait(b)
            scatter_start(g + _NBUF + b, b)
        for b in range(_NBUF):
            scatter_wait(b)

    return gather_kernel


def kernel(tensor, idx):
    V, D = tensor.shape
    (B,) = idx.shape
    b_per_w = B // _NW
    idx3 = idx.reshape(_NW, b_per_w // _CHUNK, _CHUNK)
    return _make_gather(V, D, B)(tensor, idx3)
